# baseline (speedup 1.0000x reference)
import functools

import jax
import jax.numpy as jnp
from jax import lax
from jax.experimental import pallas as pl
from jax.experimental.pallas import tpu as pltpu
from jax.experimental.pallas import tpu_sc as plsc

NUM_CORES = 2
NUM_SUBCORES = 16
NW = NUM_CORES * NUM_SUBCORES
NPIECE = 5

_mesh = plsc.VectorSubcoreMesh(core_axis_name="core", subcore_axis_name="subcore")
_sc_params = pltpu.CompilerParams(use_tc_tiling_on_sc=False)
_sc_params_nl = pltpu.CompilerParams(use_tc_tiling_on_sc=False,
                                     needs_layout_passes=False)


def _deg_kernel(n, e, npad, rpt):
  epc = e // NUM_CORES
  CHUNK = 8000
  npieces = epc // CHUNK

  @functools.partial(
      pl.kernel,
      out_type=[jax.ShapeDtypeStruct((npad, 16), jnp.float32),
                jax.ShapeDtypeStruct((npad, 16), jnp.float32)],
      mesh=_mesh,
      compiler_params=_sc_params_nl,
      scratch_types=[
          pltpu.VMEM((CHUNK,), jnp.int32),
          pltpu.VMEM((CHUNK,), jnp.int32),
          pltpu.VMEM((npad,), jnp.float32),
          pltpu.VMEM((rpt, 16), jnp.float32),
          pltpu.SemaphoreType.DMA,
          pltpu.SemaphoreType.DMA,
      ],
  )
  def k(ei_hbm, out0, out1, buf0, buf1, hist, rep, sem0, sem1):
    c = lax.axis_index("core")
    s = lax.axis_index("subcore")
    base = c * epc
    zero16 = jnp.zeros((16,), jnp.float32)
    one16 = jnp.ones((16,), jnp.float32)

    @pl.loop(0, npad // 16)
    def _(i):
      hist[pl.ds(i * 16, 16)] = zero16

    bufs = [buf0, buf1]
    sems = [sem0, sem1]
    pltpu.async_copy(ei_hbm.at[1, pl.ds(base, CHUNK)], buf0, sem0)
    pltpu.async_copy(ei_hbm.at[1, pl.ds(base + CHUNK, CHUNK)], buf1, sem1)
    for p in range(npieces):
      b = bufs[p % 2]
      pltpu.make_async_copy(
          ei_hbm.at[1, pl.ds(base + p * CHUNK, CHUNK)], b, sems[p % 2]).wait()

      @pl.loop(0, CHUNK // 16)
      def _(j):
        idx = b[pl.ds(j * 16, 16)]
        plsc.addupdate_scatter(hist, [idx], one16)

      if p + 2 < npieces:
        pltpu.async_copy(
            ei_hbm.at[1, pl.ds(base + (p + 2) * CHUNK, CHUNK)], b, sems[p % 2])

    @pl.loop(0, rpt)
    def _(i):
      full = lax.broadcast_in_dim(s * rpt + i, (16,), ())
      rep[i, :] = plsc.load_gather(hist, [full])

    @pl.when(c == 0)
    def _():
      pltpu.sync_copy(rep, out0.at[pl.ds(s * rpt, rpt)])

    @pl.when(c == 1)
    def _():
      pltpu.sync_copy(rep, out1.at[pl.ds(s * rpt, rpt)])

  return k


def _agg_kernel(n, e, npad, rpt):
  ept = e // NW
  pp = ept // NPIECE
  nr = npad // 8
  rg = rpt // 8

  @functools.partial(
      pl.kernel,
      out_type=[jax.ShapeDtypeStruct((npad, 16), jnp.float32),
                jax.ShapeDtypeStruct((npad, 16), jnp.float32)],
      mesh=_mesh,
      compiler_params=_sc_params,
      scratch_types=(
          [pltpu.VMEM((pp,), jnp.int32) for _ in range(2 * NPIECE)] + [
              pltpu.VMEM((pp, 16), jnp.float32),
              pltpu.VMEM((pp, 16), jnp.float32),
              pltpu.VMEM_SHARED((npad, 16), jnp.float32),
              pltpu.SemaphoreType.DMA,
              pltpu.SemaphoreType.DMA,
          ]
      ),
  )
  def k(t_hbm, ei_hbm, zeros_hbm, out0, out1, *refs):
    src_vs = refs[:NPIECE]
    dst_vs = refs[NPIECE:2 * NPIECE]
    bufs = refs[2 * NPIECE:2 * NPIECE + 2]
    acc = refs[2 * NPIECE + 2]
    sems = refs[2 * NPIECE + 3:2 * NPIECE + 5]
    t_tab = t_hbm
    c = lax.axis_index("core")
    s = lax.axis_index("subcore")
    w = c * NUM_SUBCORES + s
    base = w * ept
    for q in range(NPIECE):
      pltpu.sync_copy(ei_hbm.at[0, pl.ds(base + q * pp, pp)], src_vs[q])
      pltpu.sync_copy(ei_hbm.at[1, pl.ds(base + q * pp, pp)], dst_vs[q])
    pltpu.sync_copy(zeros_hbm, acc.at[pl.ds(s * rpt, rpt)])
    plsc.subcore_barrier()

    pltpu.async_copy(t_tab.at[src_vs[0]], bufs[0], sems[0])
    pltpu.async_copy(t_tab.at[src_vs[1]], bufs[1], sems[1])
    for q in range(NPIECE):
      pltpu.make_async_copy(t_tab.at[src_vs[q]], bufs[q % 2], sems[q % 2]).wait()
      pltpu.sync_copy(bufs[q % 2], acc.at[dst_vs[q]], add=True)
      if q + 2 < NPIECE:
        pltpu.async_copy(t_tab.at[src_vs[q + 2]], bufs[q % 2], sems[q % 2])

    plsc.subcore_barrier()
    src = acc.at[pl.ds(s * rpt, rpt)]

    @pl.when(c == 0)
    def _():
      pltpu.sync_copy(src, out0.at[pl.ds(s * rpt, rpt)])

    @pl.when(c == 1)
    def _():
      pltpu.sync_copy(src, out1.at[pl.ds(s * rpt, rpt)])

  return k


def kernel(x, edge_index, W1, b1, W2, b2):
  n, d = x.shape
  h = W1.shape[1]
  e = edge_index.shape[1]
  c = W2.shape[1]

  rpt = -(-(n + 1) // (NUM_SUBCORES * 64)) * 64
  npad = rpt * NUM_SUBCORES
  nr = npad // 8

  zeros16 = jnp.zeros((rpt, 16), jnp.float32)
  b1blk = jnp.tile(b1, 8).reshape(1, 8 * h)
  w2blk = jnp.kron(jnp.eye(8, dtype=jnp.float32), W2)
  b2blk = jnp.tile(b2, 8).reshape(1, 8 * c)

  deg_k = _deg_kernel(n, e, npad, rpt)
  agg_k = _agg_kernel(n, e, npad, rpt)

  bnm = 2048

  def _k_mm(x_ref, w_ref, o_ref):
    o_ref[...] = jnp.dot(x_ref[...], w_ref[...],
                         preferred_element_type=jnp.float32)

  hh = pl.pallas_call(
      _k_mm,
      grid=(npad // bnm,),
      in_specs=[pl.BlockSpec((bnm, d), lambda i: (i, 0)),
                pl.BlockSpec((d, h), lambda i: (0, 0))],
      out_specs=pl.BlockSpec((bnm, h), lambda i: (i, 0)),
      out_shape=jax.ShapeDtypeStruct((npad, h), jnp.float32),
  )(x, W1)
  hh128 = hh.reshape(nr, 8 * h)

  p0n, p1n = deg_k(edge_index)
  p0g = p0n.reshape(nr, 128)
  p1g = p1n.reshape(nr, 128)

  bne = 256
  ge = (nr // bne,)
  eb = lambda: pl.BlockSpec((bne, 128), lambda i: (i, 0))

  def _k2(p0_ref, p1_ref, hh_ref, t_ref, dis_ref):
    dis = lax.rsqrt(1.0 + p0_ref[...] + p1_ref[...])
    dis_ref[...] = dis
    t_ref[...] = hh_ref[...] * dis

  t128, dis128 = pl.pallas_call(
      _k2,
      grid=ge,
      in_specs=[eb(), eb(), eb()],
      out_specs=[eb(), eb()],
      out_shape=[jax.ShapeDtypeStruct((nr, 128), jnp.float32),
                 jax.ShapeDtypeStruct((nr, 128), jnp.float32)],
  )(p0g, p1g, hh128)

  s10n, s11n = agg_k(t128.reshape(npad, 16), edge_index, zeros16)
  s10 = s10n.reshape(nr, 128)
  s11 = s11n.reshape(nr, 128)

  def _k4(s0_ref, s1_ref, t_ref, dis_ref, b_ref, u_ref):
    agg = (s0_ref[...] + s1_ref[...] + t_ref[...]) * dis_ref[...] + b_ref[...]
    u_ref[...] = jnp.maximum(agg, 0.0) * dis_ref[...]

  u128 = pl.pallas_call(
      _k4,
      grid=ge,
      in_specs=[eb(), eb(), eb(), eb(),
                pl.BlockSpec((1, 128), lambda i: (0, 0))],
      out_specs=eb(),
      out_shape=jax.ShapeDtypeStruct((nr, 128), jnp.float32),
  )(s10, s11, t128, dis128, b1blk)

  s20n, s21n = agg_k(u128.reshape(npad, 16), edge_index, zeros16)
  s20 = s20n.reshape(nr, 128)
  s21 = s21n.reshape(nr, 128)

  def _k6(s0_ref, s1_ref, u_ref, dis_ref, w_ref, b_ref, o_ref):
    agg = (s0_ref[...] + s1_ref[...] + u_ref[...]) * dis_ref[...]
    o_ref[...] = jnp.dot(agg, w_ref[...],
                         preferred_element_type=jnp.float32) + b_ref[...]

  og = pl.pallas_call(
      _k6,
      grid=ge,
      in_specs=[eb(), eb(), eb(), eb(),
                pl.BlockSpec((128, 8 * c), lambda i: (0, 0)),
                pl.BlockSpec((1, 8 * c), lambda i: (0, 0))],
      out_specs=pl.BlockSpec((bne, 8 * c), lambda i: (i, 0)),
      out_shape=jax.ShapeDtypeStruct((n // 8, 8 * c), jnp.float32),
  )(s20, s21, u128, dis128, w2blk, b2blk)

  return og.reshape(n, c)


# speedup vs baseline: 1.2655x; 1.2655x over previous
import functools

import jax
import jax.numpy as jnp
from jax import lax
from jax.experimental import pallas as pl
from jax.experimental.pallas import tpu as pltpu
from jax.experimental.pallas import tpu_sc as plsc

NUM_CORES = 2
NUM_SUBCORES = 16
NW = NUM_CORES * NUM_SUBCORES
NPIECE = 5

_mesh = plsc.VectorSubcoreMesh(core_axis_name="core", subcore_axis_name="subcore")
_sc_params = pltpu.CompilerParams(use_tc_tiling_on_sc=False)
_sc_params_nl = pltpu.CompilerParams(use_tc_tiling_on_sc=False,
                                     needs_layout_passes=False)


def _deg_kernel(n, e, npad, rpt):
  ept = e // NW
  pp = ept // NPIECE

  @functools.partial(
      pl.kernel,
      out_type=[jax.ShapeDtypeStruct((npad, 16), jnp.float32),
                jax.ShapeDtypeStruct((npad, 16), jnp.float32)],
      mesh=_mesh,
      compiler_params=_sc_params,
      scratch_types=(
          [pltpu.VMEM((pp,), jnp.int32) for _ in range(NPIECE)] + [
              pltpu.VMEM((pp, 16), jnp.float32),
              pltpu.VMEM_SHARED((npad, 16), jnp.float32),
          ] + [pltpu.SemaphoreType.DMA for _ in range(NPIECE)]
      ),
  )
  def k(ei_hbm, ones_hbm, zeros_hbm, out0, out1, *refs):
    dst_vs = refs[:NPIECE]
    ones_v = refs[NPIECE]
    acc = refs[NPIECE + 1]
    sems = refs[NPIECE + 2:NPIECE + 2 + NPIECE]
    c = lax.axis_index("core")
    s = lax.axis_index("subcore")
    w = c * NUM_SUBCORES + s
    base = w * ept
    for q in range(NPIECE):
      pltpu.sync_copy(ei_hbm.at[1, pl.ds(base + q * pp, pp)], dst_vs[q])
    pltpu.sync_copy(ones_hbm, ones_v)
    pltpu.sync_copy(zeros_hbm, acc.at[pl.ds(s * rpt, rpt)])
    plsc.subcore_barrier()

    descs = [pltpu.async_copy(ones_v, acc.at[dst_vs[q]], sems[q], add=True)
             for q in range(NPIECE)]
    for d_ in descs:
      d_.wait()

    plsc.subcore_barrier()
    src = acc.at[pl.ds(s * rpt, rpt)]

    @pl.when(c == 0)
    def _():
      pltpu.sync_copy(src, out0.at[pl.ds(s * rpt, rpt)])

    @pl.when(c == 1)
    def _():
      pltpu.sync_copy(src, out1.at[pl.ds(s * rpt, rpt)])

  return k


def _agg_kernel(n, e, npad, rpt):
  ept = e // NW
  pp = ept // NPIECE
  nr = npad // 8
  rg = rpt // 8

  @functools.partial(
      pl.kernel,
      out_type=[jax.ShapeDtypeStruct((npad, 16), jnp.float32),
                jax.ShapeDtypeStruct((npad, 16), jnp.float32)],
      mesh=_mesh,
      compiler_params=_sc_params,
      scratch_types=(
          [pltpu.VMEM((pp,), jnp.int32) for _ in range(2 * NPIECE)] + [
              pltpu.VMEM((pp, 16), jnp.float32),
              pltpu.VMEM((pp, 16), jnp.float32),
              pltpu.VMEM_SHARED((npad, 16), jnp.float32),
              pltpu.SemaphoreType.DMA,
              pltpu.SemaphoreType.DMA,
          ]
      ),
  )
  def k(t_hbm, ei_hbm, zeros_hbm, out0, out1, *refs):
    src_vs = refs[:NPIECE]
    dst_vs = refs[NPIECE:2 * NPIECE]
    bufs = refs[2 * NPIECE:2 * NPIECE + 2]
    acc = refs[2 * NPIECE + 2]
    sems = refs[2 * NPIECE + 3:2 * NPIECE + 5]
    t_tab = t_hbm
    c = lax.axis_index("core")
    s = lax.axis_index("subcore")
    w = c * NUM_SUBCORES + s
    base = w * ept
    for q in range(NPIECE):
      pltpu.sync_copy(ei_hbm.at[0, pl.ds(base + q * pp, pp)], src_vs[q])
      pltpu.sync_copy(ei_hbm.at[1, pl.ds(base + q * pp, pp)], dst_vs[q])
    pltpu.sync_copy(zeros_hbm, acc.at[pl.ds(s * rpt, rpt)])
    plsc.subcore_barrier()

    pltpu.async_copy(t_tab.at[src_vs[0]], bufs[0], sems[0])
    pltpu.async_copy(t_tab.at[src_vs[1]], bufs[1], sems[1])
    for q in range(NPIECE):
      pltpu.make_async_copy(t_tab.at[src_vs[q]], bufs[q % 2], sems[q % 2]).wait()
      pltpu.sync_copy(bufs[q % 2], acc.at[dst_vs[q]], add=True)
      if q + 2 < NPIECE:
        pltpu.async_copy(t_tab.at[src_vs[q + 2]], bufs[q % 2], sems[q % 2])

    plsc.subcore_barrier()
    src = acc.at[pl.ds(s * rpt, rpt)]

    @pl.when(c == 0)
    def _():
      pltpu.sync_copy(src, out0.at[pl.ds(s * rpt, rpt)])

    @pl.when(c == 1)
    def _():
      pltpu.sync_copy(src, out1.at[pl.ds(s * rpt, rpt)])

  return k


def kernel(x, edge_index, W1, b1, W2, b2):
  n, d = x.shape
  h = W1.shape[1]
  e = edge_index.shape[1]
  c = W2.shape[1]

  rpt = -(-(n + 1) // (NUM_SUBCORES * 64)) * 64
  npad = rpt * NUM_SUBCORES
  nr = npad // 8

  zeros16 = jnp.zeros((rpt, 16), jnp.float32)
  b1blk = jnp.tile(b1, 8).reshape(1, 8 * h)
  w2blk = jnp.kron(jnp.eye(8, dtype=jnp.float32), W2)
  b2blk = jnp.tile(b2, 8).reshape(1, 8 * c)

  deg_k = _deg_kernel(n, e, npad, rpt)
  agg_k = _agg_kernel(n, e, npad, rpt)

  bnm = 2048

  def _k_mm(x_ref, w_ref, o_ref):
    o_ref[...] = jnp.dot(x_ref[...], w_ref[...],
                         preferred_element_type=jnp.float32)

  hh = pl.pallas_call(
      _k_mm,
      grid=(npad // bnm,),
      in_specs=[pl.BlockSpec((bnm, d), lambda i: (i, 0)),
                pl.BlockSpec((d, h), lambda i: (0, 0))],
      out_specs=pl.BlockSpec((bnm, h), lambda i: (i, 0)),
      out_shape=jax.ShapeDtypeStruct((npad, h), jnp.float32),
  )(x, W1)
  hh128 = hh.reshape(nr, 8 * h)

  ones16 = jnp.ones((e // NW // NPIECE, 16), jnp.float32)
  p0n, p1n = deg_k(edge_index, ones16, zeros16)
  p0g = p0n.reshape(nr, 128)
  p1g = p1n.reshape(nr, 128)

  bne = 256
  ge = (nr // bne,)
  eb = lambda: pl.BlockSpec((bne, 128), lambda i: (i, 0))

  def _k2(p0_ref, p1_ref, hh_ref, t_ref, dis_ref):
    dis = lax.rsqrt(1.0 + p0_ref[...] + p1_ref[...])
    dis_ref[...] = dis
    t_ref[...] = hh_ref[...] * dis

  t128, dis128 = pl.pallas_call(
      _k2,
      grid=ge,
      in_specs=[eb(), eb(), eb()],
      out_specs=[eb(), eb()],
      out_shape=[jax.ShapeDtypeStruct((nr, 128), jnp.float32),
                 jax.ShapeDtypeStruct((nr, 128), jnp.float32)],
  )(p0g, p1g, hh128)

  s10n, s11n = agg_k(t128.reshape(npad, 16), edge_index, zeros16)
  s10 = s10n.reshape(nr, 128)
  s11 = s11n.reshape(nr, 128)

  def _k4(s0_ref, s1_ref, t_ref, dis_ref, b_ref, u_ref):
    agg = (s0_ref[...] + s1_ref[...] + t_ref[...]) * dis_ref[...] + b_ref[...]
    u_ref[...] = jnp.maximum(agg, 0.0) * dis_ref[...]

  u128 = pl.pallas_call(
      _k4,
      grid=ge,
      in_specs=[eb(), eb(), eb(), eb(),
                pl.BlockSpec((1, 128), lambda i: (0, 0))],
      out_specs=eb(),
      out_shape=jax.ShapeDtypeStruct((nr, 128), jnp.float32),
  )(s10, s11, t128, dis128, b1blk)

  s20n, s21n = agg_k(u128.reshape(npad, 16), edge_index, zeros16)
  s20 = s20n.reshape(nr, 128)
  s21 = s21n.reshape(nr, 128)

  def _k6(s0_ref, s1_ref, u_ref, dis_ref, w_ref, b_ref, o_ref):
    agg = (s0_ref[...] + s1_ref[...] + u_ref[...]) * dis_ref[...]
    o_ref[...] = jnp.dot(agg, w_ref[...],
                         preferred_element_type=jnp.float32) + b_ref[...]

  og = pl.pallas_call(
      _k6,
      grid=ge,
      in_specs=[eb(), eb(), eb(), eb(),
                pl.BlockSpec((128, 8 * c), lambda i: (0, 0)),
                pl.BlockSpec((1, 8 * c), lambda i: (0, 0))],
      out_specs=pl.BlockSpec((bne, 8 * c), lambda i: (i, 0)),
      out_shape=jax.ShapeDtypeStruct((n // 8, 8 * c), jnp.float32),
  )(s20, s21, u128, dis128, w2blk, b2blk)

  return og.reshape(n, c)
